# Initial kernel scaffold; baseline (speedup 1.0000x reference)
import jax
import jax.numpy as jnp
from jax.experimental import pallas as pl


def kernel(visits, table):
    raise NotImplementedError("write your pallas kernel here")


# speedup vs baseline: 4.8217x; 4.8217x over previous
import functools

import jax
import jax.numpy as jnp
from jax import lax
from jax.experimental import pallas as pl
from jax.experimental.pallas import tpu as pltpu
from jax.experimental.pallas import tpu_sc as plsc

_VOCAB = 100000
_DIM = 128
_AUG = 144
_NUM_VISITS = 8192
_MAX_CODES = 64
_EPS = 1e-6

_NC = 2
_NS = 16
_NW = _NC * _NS
_VPW = _NUM_VISITS // _NW
_CODES_PW = _VPW * _MAX_CODES
_CH = 512
_NCHUNK = _CODES_PW // _CH
_VIS_PER_CH = _CH // _MAX_CODES


def _augment_body(tab_ref, aug_ref):
    z = tab_ref[...]
    zn2 = jnp.sum(z * z, axis=1, keepdims=True)
    k = (2.0 * z) / (1.0 + zn2)
    kn2 = jnp.sum(k * k, axis=1, keepdims=True)
    gamma = lax.rsqrt(jnp.clip(1.0 - kn2, _EPS, None))
    rows = lax.broadcasted_iota(jnp.int32, (z.shape[0], 1), 0) + pl.program_id(0) * z.shape[0]
    gamma = jnp.where(rows == 0, 0.0, gamma)
    aug_ref[:, :_DIM] = gamma * k
    aug_ref[:, _DIM:] = jnp.broadcast_to(gamma, (z.shape[0], _AUG - _DIM))


_BA = 1000


def _augment(table):
    return pl.pallas_call(
        _augment_body,
        grid=(_VOCAB // _BA,),
        in_specs=[pl.BlockSpec((_BA, _DIM), lambda i: (i, 0))],
        out_specs=pl.BlockSpec((_BA, _AUG), lambda i: (i, 0)),
        out_shape=jax.ShapeDtypeStruct((_VOCAB, _AUG), jnp.float32),
    )(table)


def _segsum_body(aug_hbm, visits_hbm, out_hbm, idx_v, rows_v, seg_v, shared):
    c = lax.axis_index("c")
    s = lax.axis_index("s")
    w = c * _NS + s

    pltpu.sync_copy(visits_hbm.at[w], idx_v)

    def _zero_row(r, _):
        for t in range(_AUG // 16):
            rows_v[r, pl.ds(t * 16, 16)] = jnp.zeros((16,), jnp.float32)
        return 0
    lax.fori_loop(0, _VPW, _zero_row, 0)
    pltpu.sync_copy(rows_v.at[pl.ds(0, _VPW)], shared.at[pl.ds(s * _VPW, _VPW)])

    def _chunk(i, _):
        pltpu.sync_copy(aug_hbm.at[idx_v.at[i]], rows_v)
        base = s * _VPW + i * _VIS_PER_CH
        for t in range(_CH // 16):
            seg_v[pl.ds(t * 16, 16)] = jnp.full((16,), base + t // 4, jnp.int32)
        pltpu.sync_copy(rows_v, shared.at[seg_v], add=True)
        return 0

    lax.fori_loop(0, _NCHUNK, _chunk, 0)

    pltpu.sync_copy(shared.at[pl.ds(s * _VPW, _VPW)],
                    out_hbm.at[pl.ds((c * _NS + s) * _VPW, _VPW)])


@functools.cache
def _segsum():
    return pl.kernel(
        _segsum_body,
        out_type=jax.ShapeDtypeStruct((_NUM_VISITS, _AUG), jnp.float32),
        mesh=plsc.VectorSubcoreMesh(core_axis_name="c", subcore_axis_name="s",
                                    num_cores=_NC, num_subcores=_NS),
        compiler_params=pltpu.CompilerParams(use_tc_tiling_on_sc=False),
        scratch_types=[
            pltpu.VMEM((_NCHUNK, _CH), jnp.int32),
            pltpu.VMEM((_CH, _AUG), jnp.float32),
            pltpu.VMEM((_CH,), jnp.int32),
            pltpu.VMEM_SHARED((_NS * _VPW, _AUG), jnp.float32),
        ],
    )


def _finish_body(s_ref, o_ref):
    acc = s_ref[...]
    S = acc[:, :_DIM]
    W = acc[:, _DIM:_DIM + 1]
    m = S / jnp.clip(W, _EPS, None)
    mn2 = jnp.sum(m * m, axis=1, keepdims=True)
    p = m / (1.0 + jnp.sqrt(jnp.clip(1.0 - mn2, _EPS, None)))
    pn = jnp.sqrt(jnp.clip(jnp.sum(p * p, axis=1, keepdims=True), _EPS, None))
    pc = jnp.clip(pn, None, 1.0 - 1e-5)
    o_ref[...] = (0.5 * jnp.log((1.0 + pc) / (1.0 - pc))) * p / pn


_BC = 512


def _finish(sums):
    return pl.pallas_call(
        _finish_body,
        grid=(_NUM_VISITS // _BC,),
        in_specs=[pl.BlockSpec((_BC, _AUG), lambda i: (i, 0))],
        out_specs=pl.BlockSpec((_BC, _DIM), lambda i: (i, 0)),
        out_shape=jax.ShapeDtypeStruct((_NUM_VISITS, _DIM), jnp.float32),
    )(sums)


def kernel(visits, table):
    aug = _augment(table)
    visits3 = visits.reshape(_NW, _NCHUNK, _CH)
    sums = _segsum()(aug, visits3)
    return _finish(sums)


# speedup vs baseline: 6.2858x; 1.3036x over previous
import functools

import jax
import jax.numpy as jnp
from jax import lax
from jax.experimental import pallas as pl
from jax.experimental.pallas import tpu as pltpu
from jax.experimental.pallas import tpu_sc as plsc

_VOCAB = 100000
_DIM = 128
_GW = 16
_NUM_VISITS = 8192
_MAX_CODES = 64
_EPS = 1e-6

_NC = 2
_NS = 16
_NW = _NC * _NS
_VPW = _NUM_VISITS // _NW
_CODES_PW = _VPW * _MAX_CODES
_CH = 256
_NCHUNK = _CODES_PW // _CH
_VIS_PER_CH = _CH // _MAX_CODES


def _augment_body(tab_ref, g_ref, gam_ref):
    z = tab_ref[...]
    zn2 = jnp.sum(z * z, axis=1, keepdims=True)
    k = (2.0 * z) / (1.0 + zn2)
    kn2 = jnp.sum(k * k, axis=1, keepdims=True)
    gamma = lax.rsqrt(jnp.clip(1.0 - kn2, _EPS, None))
    rows = lax.broadcasted_iota(jnp.int32, (z.shape[0], 1), 0) + pl.program_id(0) * z.shape[0]
    gamma = jnp.where(rows == 0, 0.0, gamma)
    g_ref[...] = gamma * k
    gam_ref[...] = jnp.broadcast_to(gamma, (z.shape[0], _GW))


_BA = 1000


def _augment(table):
    return pl.pallas_call(
        _augment_body,
        grid=(_VOCAB // _BA,),
        in_specs=[pl.BlockSpec((_BA, _DIM), lambda i: (i, 0))],
        out_specs=(pl.BlockSpec((_BA, _DIM), lambda i: (i, 0)),
                   pl.BlockSpec((_BA, _GW), lambda i: (i, 0))),
        out_shape=(jax.ShapeDtypeStruct((_VOCAB, _DIM), jnp.float32),
                   jax.ShapeDtypeStruct((_VOCAB, _GW), jnp.float32)),
    )(table)


def _segsum_body(g_hbm, gam_hbm, visits_hbm, outg_hbm, outgam_hbm,
                 idx_v, bg0, bg1, bgam0, bgam1, seg_v, shared_g, shared_gam,
                 sem0, sem1):
    c = lax.axis_index("c")
    s = lax.axis_index("s")
    w = c * _NS + s

    pltpu.sync_copy(visits_hbm.at[w], idx_v)

    def _zero_row(r, _):
        for t in range(_DIM // 16):
            bg0[r, pl.ds(t * 16, 16)] = jnp.zeros((16,), jnp.float32)
        bgam0[r, pl.ds(0, 16)] = jnp.zeros((16,), jnp.float32)
        return 0
    lax.fori_loop(0, _CH, _zero_row, 0)
    pltpu.sync_copy(bg0.at[pl.ds(0, _VPW)], shared_g.at[pl.ds(s * _VPW, _VPW)])
    pltpu.sync_copy(bgam0.at[pl.ds(0, _VPW)], shared_gam.at[pl.ds(s * _VPW, _VPW)])

    def _gather(i, bg, bgam, sem):
        pltpu.async_copy(g_hbm.at[idx_v.at[i]], bg, sem)
        pltpu.async_copy(gam_hbm.at[idx_v.at[i]], bgam, sem)

    def _wait(bg, bgam, sem):
        pltpu.make_async_copy(g_hbm.at[pl.ds(0, _CH)], bg, sem).wait()
        pltpu.make_async_copy(gam_hbm.at[pl.ds(0, _CH)], bgam, sem).wait()

    def _scatter_add(i, bg, bgam):
        base = s * _VPW + i * _VIS_PER_CH
        for t in range(_CH // 16):
            seg_v[pl.ds(t * 16, 16)] = jnp.full((16,), base + t // 4, jnp.int32)
        pltpu.sync_copy(bg, shared_g.at[seg_v], add=True)
        pltpu.sync_copy(bgam, shared_gam.at[seg_v], add=True)

    _gather(0, bg0, bgam0, sem0)

    def _two_chunks(i2, _):
        a = 2 * i2
        _gather(a + 1, bg1, bgam1, sem1)
        _wait(bg0, bgam0, sem0)
        _scatter_add(a, bg0, bgam0)

        @pl.when(i2 < _NCHUNK // 2 - 1)
        def _():
            _gather(a + 2, bg0, bgam0, sem0)
        _wait(bg1, bgam1, sem1)
        _scatter_add(a + 1, bg1, bgam1)
        return 0

    lax.fori_loop(0, _NCHUNK // 2, _two_chunks, 0)

    pltpu.sync_copy(shared_g.at[pl.ds(s * _VPW, _VPW)],
                    outg_hbm.at[pl.ds(w * _VPW, _VPW)])
    pltpu.sync_copy(shared_gam.at[pl.ds(s * _VPW, _VPW)],
                    outgam_hbm.at[pl.ds(w * _VPW, _VPW)])


@functools.cache
def _segsum():
    return pl.kernel(
        _segsum_body,
        out_type=(jax.ShapeDtypeStruct((_NUM_VISITS, _DIM), jnp.float32),
                  jax.ShapeDtypeStruct((_NUM_VISITS, _GW), jnp.float32)),
        mesh=plsc.VectorSubcoreMesh(core_axis_name="c", subcore_axis_name="s",
                                    num_cores=_NC, num_subcores=_NS),
        compiler_params=pltpu.CompilerParams(use_tc_tiling_on_sc=False),
        scratch_types=[
            pltpu.VMEM((_NCHUNK, _CH), jnp.int32),
            pltpu.VMEM((_CH, _DIM), jnp.float32),
            pltpu.VMEM((_CH, _DIM), jnp.float32),
            pltpu.VMEM((_CH, _GW), jnp.float32),
            pltpu.VMEM((_CH, _GW), jnp.float32),
            pltpu.VMEM((_CH,), jnp.int32),
            pltpu.VMEM_SHARED((_NS * _VPW, _DIM), jnp.float32),
            pltpu.VMEM_SHARED((_NS * _VPW, _GW), jnp.float32),
            pltpu.SemaphoreType.DMA,
            pltpu.SemaphoreType.DMA,
        ],
    )


def _finish_body(sg_ref, sgam_ref, o_ref):
    S = sg_ref[...]
    W = sgam_ref[:, :1]
    m = S / jnp.clip(W, _EPS, None)
    mn2 = jnp.sum(m * m, axis=1, keepdims=True)
    p = m / (1.0 + jnp.sqrt(jnp.clip(1.0 - mn2, _EPS, None)))
    pn = jnp.sqrt(jnp.clip(jnp.sum(p * p, axis=1, keepdims=True), _EPS, None))
    pc = jnp.clip(pn, None, 1.0 - 1e-5)
    o_ref[...] = (0.5 * jnp.log((1.0 + pc) / (1.0 - pc))) * p / pn


_BC = 512


def _finish(sums_g, sums_gam):
    return pl.pallas_call(
        _finish_body,
        grid=(_NUM_VISITS // _BC,),
        in_specs=[pl.BlockSpec((_BC, _DIM), lambda i: (i, 0)),
                  pl.BlockSpec((_BC, _GW), lambda i: (i, 0))],
        out_specs=pl.BlockSpec((_BC, _DIM), lambda i: (i, 0)),
        out_shape=jax.ShapeDtypeStruct((_NUM_VISITS, _DIM), jnp.float32),
    )(sums_g, sums_gam)


def kernel(visits, table):
    g_tbl, gam_tbl = _augment(table)
    visits3 = visits.reshape(_NW, _NCHUNK, _CH)
    sums_g, sums_gam = _segsum()(g_tbl, gam_tbl, visits3)
    return _finish(sums_g, sums_gam)


# speedup vs baseline: 6.7898x; 1.0802x over previous
import functools

import jax
import jax.numpy as jnp
from jax import lax
from jax.experimental import pallas as pl
from jax.experimental.pallas import tpu as pltpu
from jax.experimental.pallas import tpu_sc as plsc

_VOCAB = 100000
_DIM = 128
_NUM_VISITS = 8192
_MAX_CODES = 64
_EPS = 1e-6

_NC = 2
_NS = 16
_NW = _NC * _NS
_VPW = _NUM_VISITS // _NW
_CODES_PW = _VPW * _MAX_CODES
_CH = 256
_NCHUNK = _CODES_PW // _CH
_VIS_PER_CH = _CH // _MAX_CODES


def _augment_body(tab_ref, g_ref, gam_ref):
    z = tab_ref[...]
    zn2 = jnp.sum(z * z, axis=1, keepdims=True)
    k = (2.0 * z) / (1.0 + zn2)
    kn2 = jnp.sum(k * k, axis=1, keepdims=True)
    gamma = lax.rsqrt(jnp.clip(1.0 - kn2, _EPS, None))
    rows = lax.broadcasted_iota(jnp.int32, (z.shape[0], 1), 0) + pl.program_id(0) * z.shape[0]
    gamma = jnp.where(rows == 0, 0.0, gamma)
    g_ref[...] = gamma * k

    kn2_1 = jnp.sum(k * k, axis=1)
    gamma_1 = lax.rsqrt(jnp.clip(1.0 - kn2_1, _EPS, None))
    rows_1 = lax.broadcasted_iota(jnp.int32, (z.shape[0],), 0) + pl.program_id(0) * z.shape[0]
    gamma_1 = jnp.where(rows_1 == 0, 0.0, gamma_1)
    gam_ref[...] = gamma_1[None, None, :]


_BA = 1000


def _augment(table):
    return pl.pallas_call(
        _augment_body,
        grid=(_VOCAB // _BA,),
        in_specs=[pl.BlockSpec((_BA, _DIM), lambda i: (i, 0))],
        out_specs=(pl.BlockSpec((_BA, _DIM), lambda i: (i, 0)),
                   pl.BlockSpec((1, 1, _BA), lambda i: (i, 0, 0))),
        out_shape=(jax.ShapeDtypeStruct((_VOCAB, _DIM), jnp.float32),
                   jax.ShapeDtypeStruct((_VOCAB // _BA, 1, _BA), jnp.float32)),
    )(table)


def _segsum_body(g_hbm, gam_hbm, visits_hbm, outg_hbm, outgam_hbm,
                 idx_v, bg0, bg1, bgam0, bgam1, seg0_v, seg1_v,
                 shared_g, shared_gam, sem0, sem1, sems0, sems1):
    c = lax.axis_index("c")
    s = lax.axis_index("s")
    w = c * _NS + s

    pltpu.sync_copy(visits_hbm.at[w], idx_v)

    def _zero_row(r, _):
        for t in range(_DIM // 16):
            bg0[r, pl.ds(t * 16, 16)] = jnp.zeros((16,), jnp.float32)
        return 0
    lax.fori_loop(0, _VPW, _zero_row, 0)
    for t in range(_CH // 16):
        bgam0[pl.ds(t * 16, 16)] = jnp.zeros((16,), jnp.float32)
    pltpu.sync_copy(bg0.at[pl.ds(0, _VPW)], shared_g.at[pl.ds(s * _VPW, _VPW)])
    pltpu.sync_copy(bgam0.at[pl.ds(0, _VPW)], shared_gam.at[pl.ds(s * _VPW, _VPW)])

    def _gather(i, bg, bgam, sem):
        pltpu.async_copy(g_hbm.at[idx_v.at[i]], bg, sem)
        pltpu.async_copy(gam_hbm.at[idx_v.at[i]], bgam, sem)

    def _wait_gather(bg, bgam, sem):
        pltpu.make_async_copy(g_hbm.at[pl.ds(0, _CH)], bg, sem).wait()
        pltpu.make_async_copy(gam_hbm.at[pl.ds(0, _CH)], bgam, sem).wait()

    def _scatter_add(i, bg, bgam, seg, sems):
        base = s * _VPW + i * _VIS_PER_CH
        for t in range(_CH // 16):
            seg[pl.ds(t * 16, 16)] = jnp.full((16,), base + t // 4, jnp.int32)
        pltpu.async_copy(bg, shared_g.at[seg], sems, add=True)
        pltpu.async_copy(bgam, shared_gam.at[seg], sems, add=True)

    def _wait_scatter(bg, bgam, seg, sems):
        pltpu.make_async_copy(bg, shared_g.at[seg], sems).wait()
        pltpu.make_async_copy(bgam, shared_gam.at[seg], sems).wait()

    _gather(0, bg0, bgam0, sem0)

    def _two_chunks(i2, _):
        a = 2 * i2

        @pl.when(i2 > 0)
        def _():
            _wait_scatter(bg1, bgam1, seg1_v, sems1)
        _gather(a + 1, bg1, bgam1, sem1)
        _wait_gather(bg0, bgam0, sem0)
        _scatter_add(a, bg0, bgam0, seg0_v, sems0)

        @pl.when(i2 < _NCHUNK // 2 - 1)
        def _():
            _wait_scatter(bg0, bgam0, seg0_v, sems0)
            _gather(a + 2, bg0, bgam0, sem0)
        _wait_gather(bg1, bgam1, sem1)
        _scatter_add(a + 1, bg1, bgam1, seg1_v, sems1)
        return 0

    lax.fori_loop(0, _NCHUNK // 2, _two_chunks, 0)
    _wait_scatter(bg0, bgam0, seg0_v, sems0)
    _wait_scatter(bg1, bgam1, seg1_v, sems1)

    pltpu.sync_copy(shared_g.at[pl.ds(s * _VPW, _VPW)],
                    outg_hbm.at[pl.ds(w * _VPW, _VPW)])
    pltpu.sync_copy(shared_gam.at[pl.ds(s * _VPW, _VPW)],
                    outgam_hbm.at[pl.ds(w * _VPW, _VPW)])


@functools.cache
def _segsum():
    return pl.kernel(
        _segsum_body,
        out_type=(jax.ShapeDtypeStruct((_NUM_VISITS, _DIM), jnp.float32),
                  jax.ShapeDtypeStruct((_NUM_VISITS,), jnp.float32)),
        mesh=plsc.VectorSubcoreMesh(core_axis_name="c", subcore_axis_name="s",
                                    num_cores=_NC, num_subcores=_NS),
        compiler_params=pltpu.CompilerParams(use_tc_tiling_on_sc=False),
        scratch_types=[
            pltpu.VMEM((_NCHUNK, _CH), jnp.int32),
            pltpu.VMEM((_CH, _DIM), jnp.float32),
            pltpu.VMEM((_CH, _DIM), jnp.float32),
            pltpu.VMEM((_CH,), jnp.float32),
            pltpu.VMEM((_CH,), jnp.float32),
            pltpu.VMEM((_CH,), jnp.int32),
            pltpu.VMEM((_CH,), jnp.int32),
            pltpu.VMEM_SHARED((_NS * _VPW, _DIM), jnp.float32),
            pltpu.VMEM_SHARED((_NS * _VPW,), jnp.float32),
            pltpu.SemaphoreType.DMA,
            pltpu.SemaphoreType.DMA,
            pltpu.SemaphoreType.DMA,
            pltpu.SemaphoreType.DMA,
        ],
    )


def _finish_body(sg_ref, sgam_ref, o_ref):
    S = sg_ref[...]
    W = sgam_ref[...]
    m = S / jnp.clip(W, _EPS, None)
    mn2 = jnp.sum(m * m, axis=1, keepdims=True)
    p = m / (1.0 + jnp.sqrt(jnp.clip(1.0 - mn2, _EPS, None)))
    pn = jnp.sqrt(jnp.clip(jnp.sum(p * p, axis=1, keepdims=True), _EPS, None))
    pc = jnp.clip(pn, None, 1.0 - 1e-5)
    o_ref[...] = (0.5 * jnp.log((1.0 + pc) / (1.0 - pc))) * p / pn


_BC = 512


def _finish(sums_g, sums_gam):
    return pl.pallas_call(
        _finish_body,
        grid=(_NUM_VISITS // _BC,),
        in_specs=[pl.BlockSpec((_BC, _DIM), lambda i: (i, 0)),
                  pl.BlockSpec((_BC, 1), lambda i: (i, 0))],
        out_specs=pl.BlockSpec((_BC, _DIM), lambda i: (i, 0)),
        out_shape=jax.ShapeDtypeStruct((_NUM_VISITS, _DIM), jnp.float32),
    )(sums_g, sums_gam)


def kernel(visits, table):
    g_tbl, gam2 = _augment(table)
    gam_tbl = gam2.reshape(_VOCAB)
    visits3 = visits.reshape(_NW, _NCHUNK, _CH)
    sums_g, sums_gam = _segsum()(g_tbl, gam_tbl, visits3)
    return _finish(sums_g, sums_gam[:, None])


# speedup vs baseline: 10.2686x; 1.5124x over previous
import functools

import jax
import jax.numpy as jnp
from jax import lax
from jax.experimental import pallas as pl
from jax.experimental.pallas import tpu as pltpu
from jax.experimental.pallas import tpu_sc as plsc

_VOCAB = 100000
_DIM = 128
_NUM_VISITS = 8192
_MAX_CODES = 64
_EPS = 1e-6

_NC = 2
_NS = 16
_NW = _NC * _NS
_VPW = _NUM_VISITS // _NW
_CODES_PW = _VPW * _MAX_CODES
_CH = 256
_NCHUNK = _CODES_PW // _CH
_VIS_PER_CH = _CH // _MAX_CODES
_NL = 16


def _augment_body(tab_ref, g_ref, gam_ref):
    z = tab_ref[...]
    zn2 = jnp.sum(z * z, axis=1, keepdims=True)
    k = (2.0 * z) / (1.0 + zn2)
    kn2 = jnp.sum(k * k, axis=1, keepdims=True)
    gamma = lax.rsqrt(jnp.clip(1.0 - kn2, _EPS, None))
    rows = lax.broadcasted_iota(jnp.int32, (z.shape[0], 1), 0) + pl.program_id(0) * z.shape[0]
    gamma = jnp.where(rows == 0, 0.0, gamma)
    g_ref[...] = gamma * k

    kn2_1 = jnp.sum(k * k, axis=1)
    gamma_1 = lax.rsqrt(jnp.clip(1.0 - kn2_1, _EPS, None))
    rows_1 = lax.broadcasted_iota(jnp.int32, (z.shape[0],), 0) + pl.program_id(0) * z.shape[0]
    gamma_1 = jnp.where(rows_1 == 0, 0.0, gamma_1)
    gam_ref[...] = gamma_1[None, None, :]


_BA = 2000


def _augment(table):
    return pl.pallas_call(
        _augment_body,
        grid=(_VOCAB // _BA,),
        in_specs=[pl.BlockSpec((_BA, _DIM), lambda i: (i, 0))],
        out_specs=(pl.BlockSpec((_BA, _DIM), lambda i: (i, 0)),
                   pl.BlockSpec((1, 1, _BA), lambda i: (i, 0, 0))),
        out_shape=(jax.ShapeDtypeStruct((_VOCAB, _DIM), jnp.float32),
                   jax.ShapeDtypeStruct((_VOCAB // _BA, 1, _BA), jnp.float32)),
    )(table)


def _segsum_body(g_hbm, gam_hbm, visits_hbm, outg_hbm, outgam_hbm,
                 idx_v, bg0, bg1, bgam0, bgam1, obuf, wbuf, sem0, sem1):
    c = lax.axis_index("c")
    s = lax.axis_index("s")
    w = c * _NS + s

    pltpu.sync_copy(visits_hbm.at[w], idx_v)

    def _gather(i, bg, bgam, sem):
        pltpu.async_copy(g_hbm.at[idx_v.at[i]], bg, sem)
        pltpu.async_copy(gam_hbm.at[idx_v.at[i]], bgam, sem)

    def _wait_gather(bg, bgam, sem):
        pltpu.make_async_copy(g_hbm.at[pl.ds(0, _CH)], bg, sem).wait()
        pltpu.make_async_copy(gam_hbm.at[pl.ds(0, _CH)], bgam, sem).wait()

    def _process(i, bg, bgam):
        for v in range(_VIS_PER_CH):
            vis = i * _VIS_PER_CH + v

            def _rows(r4, accs):
                out = list(accs)
                for u in range(4):
                    row = v * _MAX_CODES + r4 * 4 + u
                    for t in range(_DIM // _NL):
                        out[t] = out[t] + bg[row, pl.ds(t * _NL, _NL)]
                return tuple(out)

            accs = lax.fori_loop(
                0, _MAX_CODES // 4, _rows,
                tuple(jnp.zeros((_NL,), jnp.float32) for _ in range(_DIM // _NL)))
            for t in range(_DIM // _NL):
                obuf[vis, pl.ds(t * _NL, _NL)] = accs[t]
            wsum = (bgam[pl.ds(v * _MAX_CODES, _NL)]
                    + bgam[pl.ds(v * _MAX_CODES + _NL, _NL)]
                    + bgam[pl.ds(v * _MAX_CODES + 2 * _NL, _NL)]
                    + bgam[pl.ds(v * _MAX_CODES + 3 * _NL, _NL)])
            wbuf[vis, pl.ds(0, _NL)] = wsum

    _gather(0, bg0, bgam0, sem0)

    def _two_chunks(i2, _):
        a = 2 * i2
        _gather(a + 1, bg1, bgam1, sem1)
        _wait_gather(bg0, bgam0, sem0)
        _process(a, bg0, bgam0)

        @pl.when(i2 < _NCHUNK // 2 - 1)
        def _():
            _gather(a + 2, bg0, bgam0, sem0)
        _wait_gather(bg1, bgam1, sem1)
        _process(a + 1, bg1, bgam1)
        return 0

    lax.fori_loop(0, _NCHUNK // 2, _two_chunks, 0)

    pltpu.sync_copy(obuf, outg_hbm.at[pl.ds(w * _VPW, _VPW)])
    pltpu.sync_copy(wbuf, outgam_hbm.at[pl.ds(w * _VPW, _VPW)])


@functools.cache
def _segsum():
    return pl.kernel(
        _segsum_body,
        out_type=(jax.ShapeDtypeStruct((_NUM_VISITS, _DIM), jnp.float32),
                  jax.ShapeDtypeStruct((_NUM_VISITS, _NL), jnp.float32)),
        mesh=plsc.VectorSubcoreMesh(core_axis_name="c", subcore_axis_name="s",
                                    num_cores=_NC, num_subcores=_NS),
        compiler_params=pltpu.CompilerParams(use_tc_tiling_on_sc=False),
        scratch_types=[
            pltpu.VMEM((_NCHUNK, _CH), jnp.int32),
            pltpu.VMEM((_CH, _DIM), jnp.float32),
            pltpu.VMEM((_CH, _DIM), jnp.float32),
            pltpu.VMEM((_CH,), jnp.float32),
            pltpu.VMEM((_CH,), jnp.float32),
            pltpu.VMEM((_VPW, _DIM), jnp.float32),
            pltpu.VMEM((_VPW, _NL), jnp.float32),
            pltpu.SemaphoreType.DMA,
            pltpu.SemaphoreType.DMA,
        ],
    )


def _finish_body(sg_ref, sgam_ref, o_ref):
    S = sg_ref[...]
    W = jnp.sum(sgam_ref[...], axis=1, keepdims=True)
    m = S / jnp.clip(W, _EPS, None)
    mn2 = jnp.sum(m * m, axis=1, keepdims=True)
    p = m / (1.0 + jnp.sqrt(jnp.clip(1.0 - mn2, _EPS, None)))
    pn = jnp.sqrt(jnp.clip(jnp.sum(p * p, axis=1, keepdims=True), _EPS, None))
    pc = jnp.clip(pn, None, 1.0 - 1e-5)
    o_ref[...] = (0.5 * jnp.log((1.0 + pc) / (1.0 - pc))) * p / pn


_BC = 512


def _finish(sums_g, sums_gam):
    return pl.pallas_call(
        _finish_body,
        grid=(_NUM_VISITS // _BC,),
        in_specs=[pl.BlockSpec((_BC, _DIM), lambda i: (i, 0)),
                  pl.BlockSpec((_BC, _NL), lambda i: (i, 0))],
        out_specs=pl.BlockSpec((_BC, _DIM), lambda i: (i, 0)),
        out_shape=jax.ShapeDtypeStruct((_NUM_VISITS, _DIM), jnp.float32),
    )(sums_g, sums_gam)


def kernel(visits, table):
    g_tbl, gam2 = _augment(table)
    gam_tbl = gam2.reshape(_VOCAB)
    visits3 = visits.reshape(_NW, _NCHUNK, _CH)
    sums_g, sums_gam = _segsum()(g_tbl, gam_tbl, visits3)
    return _finish(sums_g, sums_gam)


# speedup vs baseline: 10.8050x; 1.0522x over previous
import functools

import jax
import jax.numpy as jnp
from jax import lax
from jax.experimental import pallas as pl
from jax.experimental.pallas import tpu as pltpu
from jax.experimental.pallas import tpu_sc as plsc

_VOCAB = 100000
_DIM = 128
_NUM_VISITS = 8192
_MAX_CODES = 64
_EPS = 1e-6

_NC = 2
_NS = 16
_NW = _NC * _NS
_VPW = _NUM_VISITS // _NW
_CODES_PW = _VPW * _MAX_CODES
_CH = 256
_NCHUNK = _CODES_PW // _CH
_VIS_PER_CH = _CH // _MAX_CODES
_NL = 16


def _augment_body(tab_ref, g_ref, gam_ref):
    z = tab_ref[...]
    zn2 = jnp.sum(z * z, axis=1, keepdims=True)
    k = (2.0 * z) / (1.0 + zn2)
    kn2 = jnp.sum(k * k, axis=1, keepdims=True)
    gamma = lax.rsqrt(jnp.clip(1.0 - kn2, _EPS, None))
    rows = lax.broadcasted_iota(jnp.int32, (z.shape[0], 1), 0) + pl.program_id(0) * z.shape[0]
    gamma = jnp.where(rows == 0, 0.0, gamma)
    g_ref[...] = gamma * k

    kn2_1 = jnp.sum(k * k, axis=1)
    gamma_1 = lax.rsqrt(jnp.clip(1.0 - kn2_1, _EPS, None))
    rows_1 = lax.broadcasted_iota(jnp.int32, (z.shape[0],), 0) + pl.program_id(0) * z.shape[0]
    gamma_1 = jnp.where(rows_1 == 0, 0.0, gamma_1)
    gam_ref[...] = gamma_1[None, None, :]


_BA = 4000


def _augment(table):
    return pl.pallas_call(
        _augment_body,
        grid=(_VOCAB // _BA,),
        in_specs=[pl.BlockSpec((_BA, _DIM), lambda i: (i, 0))],
        out_specs=(pl.BlockSpec((_BA, _DIM), lambda i: (i, 0)),
                   pl.BlockSpec((1, 1, _BA), lambda i: (i, 0, 0))),
        out_shape=(jax.ShapeDtypeStruct((_VOCAB, _DIM), jnp.float32),
                   jax.ShapeDtypeStruct((_VOCAB // _BA, 1, _BA), jnp.float32)),
    )(table)


def _segsum_body(g_hbm, gam_hbm, visits_hbm, outg_hbm, outgam_hbm,
                 idx_v, bg0, bg1, bgam0, bgam1, obuf, wbuf, sem0, sem1):
    c = lax.axis_index("c")
    s = lax.axis_index("s")
    w = c * _NS + s

    pltpu.sync_copy(visits_hbm.at[w], idx_v)

    def _gather(i, bg, bgam, sem):
        pltpu.async_copy(g_hbm.at[idx_v.at[i]], bg, sem)
        pltpu.async_copy(gam_hbm.at[idx_v.at[i]], bgam, sem)

    def _wait_gather(bg, bgam, sem):
        pltpu.make_async_copy(g_hbm.at[pl.ds(0, _CH)], bg, sem).wait()
        pltpu.make_async_copy(gam_hbm.at[pl.ds(0, _CH)], bgam, sem).wait()

    def _process(i, bg, bgam):
        for v in range(_VIS_PER_CH):
            vis = i * _VIS_PER_CH + v

            def _rows(r8, accs):
                out = list(accs)
                for u in range(8):
                    row = v * _MAX_CODES + r8 * 8 + u
                    for t in range(_DIM // _NL):
                        out[t] = out[t] + bg[row, pl.ds(t * _NL, _NL)]
                return tuple(out)

            accs = lax.fori_loop(
                0, _MAX_CODES // 8, _rows,
                tuple(jnp.zeros((_NL,), jnp.float32) for _ in range(_DIM // _NL)))
            for t in range(_DIM // _NL):
                obuf[vis, pl.ds(t * _NL, _NL)] = accs[t]
            wsum = (bgam[pl.ds(v * _MAX_CODES, _NL)]
                    + bgam[pl.ds(v * _MAX_CODES + _NL, _NL)]
                    + bgam[pl.ds(v * _MAX_CODES + 2 * _NL, _NL)]
                    + bgam[pl.ds(v * _MAX_CODES + 3 * _NL, _NL)])
            wbuf[vis, pl.ds(0, _NL)] = wsum

    _gather(0, bg0, bgam0, sem0)

    def _two_chunks(i2, _):
        a = 2 * i2
        _gather(a + 1, bg1, bgam1, sem1)
        _wait_gather(bg0, bgam0, sem0)
        _process(a, bg0, bgam0)

        @pl.when(i2 < _NCHUNK // 2 - 1)
        def _():
            _gather(a + 2, bg0, bgam0, sem0)
        _wait_gather(bg1, bgam1, sem1)
        _process(a + 1, bg1, bgam1)
        return 0

    lax.fori_loop(0, _NCHUNK // 2, _two_chunks, 0)

    pltpu.sync_copy(obuf, outg_hbm.at[pl.ds(w * _VPW, _VPW)])
    pltpu.sync_copy(wbuf, outgam_hbm.at[pl.ds(w * _VPW, _VPW)])


@functools.cache
def _segsum():
    return pl.kernel(
        _segsum_body,
        out_type=(jax.ShapeDtypeStruct((_NUM_VISITS, _DIM), jnp.float32),
                  jax.ShapeDtypeStruct((_NUM_VISITS, _NL), jnp.float32)),
        mesh=plsc.VectorSubcoreMesh(core_axis_name="c", subcore_axis_name="s",
                                    num_cores=_NC, num_subcores=_NS),
        compiler_params=pltpu.CompilerParams(use_tc_tiling_on_sc=False),
        scratch_types=[
            pltpu.VMEM((_NCHUNK, _CH), jnp.int32),
            pltpu.VMEM((_CH, _DIM), jnp.float32),
            pltpu.VMEM((_CH, _DIM), jnp.float32),
            pltpu.VMEM((_CH,), jnp.float32),
            pltpu.VMEM((_CH,), jnp.float32),
            pltpu.VMEM((_VPW, _DIM), jnp.float32),
            pltpu.VMEM((_VPW, _NL), jnp.float32),
            pltpu.SemaphoreType.DMA,
            pltpu.SemaphoreType.DMA,
        ],
    )


def _finish_body(sg_ref, sgam_ref, o_ref):
    S = sg_ref[...]
    W = jnp.sum(sgam_ref[...], axis=1, keepdims=True)
    m = S / jnp.clip(W, _EPS, None)
    mn2 = jnp.sum(m * m, axis=1, keepdims=True)
    p = m / (1.0 + jnp.sqrt(jnp.clip(1.0 - mn2, _EPS, None)))
    pn = jnp.sqrt(jnp.clip(jnp.sum(p * p, axis=1, keepdims=True), _EPS, None))
    pc = jnp.clip(pn, None, 1.0 - 1e-5)
    o_ref[...] = (0.5 * jnp.log((1.0 + pc) / (1.0 - pc))) * p / pn


_BC = 512


def _finish(sums_g, sums_gam):
    return pl.pallas_call(
        _finish_body,
        grid=(_NUM_VISITS // _BC,),
        in_specs=[pl.BlockSpec((_BC, _DIM), lambda i: (i, 0)),
                  pl.BlockSpec((_BC, _NL), lambda i: (i, 0))],
        out_specs=pl.BlockSpec((_BC, _DIM), lambda i: (i, 0)),
        out_shape=jax.ShapeDtypeStruct((_NUM_VISITS, _DIM), jnp.float32),
    )(sums_g, sums_gam)


def kernel(visits, table):
    g_tbl, gam2 = _augment(table)
    gam_tbl = gam2.reshape(_VOCAB)
    visits3 = visits.reshape(_NW, _NCHUNK, _CH)
    sums_g, sums_gam = _segsum()(g_tbl, gam_tbl, visits3)
    return _finish(sums_g, sums_gam)
